# baseline (speedup 1.0000x reference)
import functools

import jax
import jax.numpy as jnp
from jax import lax
from jax.experimental import pallas as pl
from jax.experimental.pallas import tpu as pltpu
from jax.experimental.pallas import tpu_sc as plsc

N = 10000
F_IN = 128
HID = 16
NCLASS = 40
E = 320000

CHUNK = 128
E_ROWS = 2560
E_PAD = E_ROWS * CHUNK
N_PAD = 10240
NUM_TILES = 32
ROWS_PER_TILE = E_ROWS // NUM_TILES
NROWS_PER_TILE = N_PAD // 16
NH = 4
BKB = 10
NBATCH = ROWS_PER_TILE // BKB


def _sc_mesh():
    return plsc.VectorSubcoreMesh(core_axis_name="c", subcore_axis_name="s")


def _sc_degree(dst2d, zeros1):
    @functools.partial(
        pl.kernel,
        mesh=_sc_mesh(),
        compiler_params=pltpu.CompilerParams(use_tc_tiling_on_sc=False),
        out_type=jax.ShapeDtypeStruct((2, N_PAD), jnp.float32),
        scratch_types=[
            pltpu.VMEM((ROWS_PER_TILE, CHUNK), jnp.int32),
            pltpu.VMEM((CHUNK,), jnp.float32),
            pltpu.VMEM_SHARED((N_PAD,), jnp.float32),
            pltpu.SemaphoreType.DMA,
            pltpu.SemaphoreType.DMA,
        ],
    )
    def k(dst_hbm, z_hbm, out_hbm, idx_v, ones_v, acc, zsem, ssem):
        c = lax.axis_index("c")
        s = lax.axis_index("s")
        wid = s * 2 + c
        for i in range(CHUNK // 16):
            ones_v[pl.ds(i * 16, 16)] = jnp.ones((16,), jnp.float32)
        nslice = pl.ds(s * NROWS_PER_TILE, NROWS_PER_TILE)
        pltpu.async_copy(z_hbm.at[nslice], acc.at[nslice], zsem)
        pltpu.sync_copy(dst_hbm.at[pl.ds(wid * ROWS_PER_TILE, ROWS_PER_TILE)], idx_v)
        pltpu.make_async_copy(z_hbm.at[nslice], acc.at[nslice], zsem).wait()
        plsc.subcore_barrier()

        def fire(i, _):
            pltpu.async_copy(ones_v, acc.at[idx_v.at[i]], ssem, add=True)
            return 0

        lax.fori_loop(0, ROWS_PER_TILE, fire, 0)

        def drain(i, _):
            pltpu.make_async_copy(dst_hbm.at[0], idx_v.at[0], ssem).wait()
            return 0

        lax.fori_loop(0, ROWS_PER_TILE, drain, 0)
        plsc.subcore_barrier()
        pltpu.sync_copy(acc.at[nslice], out_hbm.at[c, nslice])

    return k(dst2d, zeros1)


def _sc_scatter16(table, src2d, dst2d, zeros16):
    @functools.partial(
        pl.kernel,
        mesh=_sc_mesh(),
        compiler_params=pltpu.CompilerParams(use_tc_tiling_on_sc=False),
        out_type=jax.ShapeDtypeStruct((2, N_PAD, HID), jnp.float32),
        scratch_types=[
            pltpu.VMEM((ROWS_PER_TILE, CHUNK), jnp.int32),
            pltpu.VMEM((ROWS_PER_TILE, CHUNK), jnp.int32),
            pltpu.VMEM((NH, BKB, CHUNK, HID), jnp.float32),
            pltpu.VMEM_SHARED((N_PAD, HID), jnp.float32),
            pltpu.SemaphoreType.DMA,
            [pltpu.SemaphoreType.DMA] * NH,
            [pltpu.SemaphoreType.DMA] * NH,
        ],
    )
    def k(tab_hbm, src_hbm, dst_hbm, z_hbm, out_hbm,
          src_v, dst_v, rows_v, acc, zsem, gsems, ssems):
        c = lax.axis_index("c")
        s = lax.axis_index("s")
        wid = s * 2 + c
        nslice = pl.ds(s * NROWS_PER_TILE, NROWS_PER_TILE)
        pltpu.async_copy(z_hbm.at[nslice], acc.at[nslice], zsem)
        pltpu.sync_copy(src_hbm.at[pl.ds(wid * ROWS_PER_TILE, ROWS_PER_TILE)], src_v)
        pltpu.sync_copy(dst_hbm.at[pl.ds(wid * ROWS_PER_TILE, ROWS_PER_TILE)], dst_v)
        pltpu.make_async_copy(z_hbm.at[nslice], acc.at[nslice], zsem).wait()
        plsc.subcore_barrier()

        def fire_gathers(m, h):
            def gb(j, _):
                pltpu.async_copy(
                    tab_hbm.at[src_v.at[m * BKB + j]], rows_v.at[h, j], gsems[h]
                )
                return 0
            lax.fori_loop(0, BKB, gb, 0)

        def fire_scatters(m, h):
            def sb(j, _):
                pltpu.async_copy(
                    rows_v.at[h, j], acc.at[dst_v.at[m * BKB + j]],
                    ssems[h], add=True,
                )
                return 0
            lax.fori_loop(0, BKB, sb, 0)

        def drain(sem):
            def db(j, _):
                pltpu.make_async_copy(
                    tab_hbm.at[pl.ds(0, CHUNK)], rows_v.at[0, 0], sem
                ).wait()
                return 0
            lax.fori_loop(0, BKB, db, 0)

        for h in range(NH):
            fire_gathers(h, h)
        for m in range(NBATCH):
            h = m % NH
            drain(gsems[h])
            fire_scatters(m, h)
            drain(ssems[h])
            if m + NH < NBATCH:
                fire_gathers(m + NH, h)
        plsc.subcore_barrier()
        pltpu.sync_copy(acc.at[nslice], out_hbm.at[c, nslice])

    return k(table, src2d, dst2d, zeros16)


def _tc_pre(x, W1, deg_a, deg_b):
    def body(x_ref, w_ref, da_ref, db_ref, hp_ref, dinv_ref):
        dinv = lax.rsqrt(da_ref[...] + db_ref[...] + 1.0)
        h = jnp.dot(x_ref[...], w_ref[...], preferred_element_type=jnp.float32)
        hp_ref[pl.ds(0, N), :] = h * dinv[:N]
        hp_ref[pl.ds(N, N_PAD - N), :] = jnp.zeros((N_PAD - N, HID), jnp.float32)
        dinv_ref[...] = dinv

    return pl.pallas_call(
        body,
        out_shape=[
            jax.ShapeDtypeStruct((N_PAD, HID), jnp.float32),
            jax.ShapeDtypeStruct((N_PAD, 1), jnp.float32),
        ],
    )(x, W1, deg_a, deg_b)


def _tc_mid(s1a, s1b, hp1, dinv, b1):
    def body(sa_ref, sb_ref, hp_ref, di_ref, b_ref, g_ref):
        z = di_ref[...] * (sa_ref[...] + sb_ref[...] + hp_ref[...]) + b_ref[...]
        g_ref[...] = di_ref[...] * jnp.maximum(z, 0.0)

    return pl.pallas_call(
        body,
        out_shape=jax.ShapeDtypeStruct((N_PAD, HID), jnp.float32),
    )(s1a, s1b, hp1, dinv, b1)


def _tc_out(s2a, s2b, g, dinv, W2, b2):
    def body(sa_ref, sb_ref, g_ref, di_ref, w_ref, b_ref, out_ref):
        t = di_ref[...] * (sa_ref[...] + sb_ref[...] + g_ref[...])
        o = jnp.dot(t, w_ref[...], preferred_element_type=jnp.float32) + b_ref[...]
        m = jnp.max(o, axis=1, keepdims=True)
        lse = m + jnp.log(jnp.sum(jnp.exp(o - m), axis=1, keepdims=True))
        out_ref[...] = o - lse

    return pl.pallas_call(
        body,
        out_shape=jax.ShapeDtypeStruct((N_PAD, NCLASS), jnp.float32),
    )(s2a, s2b, g, dinv, W2, b2)


def kernel(x, adj, W1, b1, W2, b2):
    src, dst = adj[0], adj[1]
    src2d = jnp.concatenate(
        [src, jnp.zeros((E_PAD - E,), jnp.int32)]
    ).reshape(E_ROWS, CHUNK)
    dst2d = jnp.concatenate(
        [dst, jnp.full((E_PAD - E,), N, jnp.int32)]
    ).reshape(E_ROWS, CHUNK)
    z16 = jnp.zeros((N_PAD, HID), jnp.float32)
    z1 = jnp.zeros((N_PAD,), jnp.float32)

    deg2 = _sc_degree(dst2d, z1)
    hp1, dinv = _tc_pre(x, W1, deg2[0][:, None], deg2[1][:, None])
    s1 = _sc_scatter16(hp1, src2d, dst2d, z16)
    g = _tc_mid(s1[0], s1[1], hp1, dinv, b1[None, :])
    s2 = _sc_scatter16(g, src2d, dst2d, z16)
    out = _tc_out(s2[0], s2[1], g, dinv, W2, b2[None, :])
    return out[:N]


# speedup vs baseline: 1.6698x; 1.6698x over previous
import functools

import jax
import jax.numpy as jnp
from jax import lax
from jax.experimental import pallas as pl
from jax.experimental.pallas import tpu as pltpu
from jax.experimental.pallas import tpu_sc as plsc

N = 10000
F_IN = 128
HID = 16
NCLASS = 40
E = 320000

CHUNK = 128
E_ROWS = 2560
E_PAD = E_ROWS * CHUNK
N_PAD = 10240
NUM_TILES = 32
ROWS_PER_TILE = E_ROWS // NUM_TILES
NROWS_PER_TILE = N_PAD // 16
NH = 4
BKB = 10
NBATCH = ROWS_PER_TILE // BKB


def _sc_mesh():
    return plsc.VectorSubcoreMesh(core_axis_name="c", subcore_axis_name="s")


def _sc_degree(dst2d, zeros1):
    @functools.partial(
        pl.kernel,
        mesh=_sc_mesh(),
        compiler_params=pltpu.CompilerParams(use_tc_tiling_on_sc=False),
        out_type=jax.ShapeDtypeStruct((2, N_PAD), jnp.float32),
        scratch_types=[
            pltpu.VMEM((ROWS_PER_TILE, CHUNK), jnp.int32),
            pltpu.VMEM((CHUNK,), jnp.float32),
            pltpu.VMEM_SHARED((N_PAD,), jnp.float32),
            pltpu.SemaphoreType.DMA,
            pltpu.SemaphoreType.DMA,
        ],
    )
    def k(dst_hbm, z_hbm, out_hbm, idx_v, ones_v, acc, zsem, ssem):
        c = lax.axis_index("c")
        s = lax.axis_index("s")
        wid = s * 2 + c
        for i in range(CHUNK // 16):
            ones_v[pl.ds(i * 16, 16)] = jnp.ones((16,), jnp.float32)
        nslice = pl.ds(s * NROWS_PER_TILE, NROWS_PER_TILE)
        pltpu.async_copy(z_hbm.at[nslice], acc.at[nslice], zsem)
        pltpu.sync_copy(dst_hbm.at[pl.ds(wid * ROWS_PER_TILE, ROWS_PER_TILE)], idx_v)
        pltpu.make_async_copy(z_hbm.at[nslice], acc.at[nslice], zsem).wait()
        plsc.subcore_barrier()

        def fire(i, _):
            pltpu.async_copy(ones_v, acc.at[idx_v.at[i]], ssem, add=True)
            return 0

        lax.fori_loop(0, ROWS_PER_TILE, fire, 0)

        def drain(i, _):
            pltpu.make_async_copy(dst_hbm.at[0], idx_v.at[0], ssem).wait()
            return 0

        lax.fori_loop(0, ROWS_PER_TILE, drain, 0)
        plsc.subcore_barrier()
        pltpu.sync_copy(acc.at[nslice], out_hbm.at[c, nslice])

    return k(dst2d, zeros1)


def _sc_scatter16(table, src2d, dst2d, zeros16):
    @functools.partial(
        pl.kernel,
        mesh=_sc_mesh(),
        compiler_params=pltpu.CompilerParams(use_tc_tiling_on_sc=False),
        out_type=jax.ShapeDtypeStruct((2, N_PAD, HID), jnp.float32),
        scratch_types=[
            pltpu.VMEM((ROWS_PER_TILE, CHUNK), jnp.int32),
            pltpu.VMEM((ROWS_PER_TILE, CHUNK), jnp.int32),
            pltpu.VMEM((NH, BKB, CHUNK, HID), jnp.float32),
            pltpu.VMEM_SHARED((N_PAD, HID), jnp.float32),
            pltpu.SemaphoreType.DMA,
            [pltpu.SemaphoreType.DMA] * NH,
            [pltpu.SemaphoreType.DMA] * NH,
        ],
    )
    def k(tab_hbm, src_hbm, dst_hbm, z_hbm, out_hbm,
          src_v, dst_v, rows_v, acc, zsem, gsems, ssems):
        c = lax.axis_index("c")
        s = lax.axis_index("s")
        wid = s * 2 + c
        nslice = pl.ds(s * NROWS_PER_TILE, NROWS_PER_TILE)
        pltpu.async_copy(z_hbm.at[nslice], acc.at[nslice], zsem)
        pltpu.sync_copy(src_hbm.at[pl.ds(wid * ROWS_PER_TILE, ROWS_PER_TILE)], src_v)
        pltpu.sync_copy(dst_hbm.at[pl.ds(wid * ROWS_PER_TILE, ROWS_PER_TILE)], dst_v)
        pltpu.make_async_copy(z_hbm.at[nslice], acc.at[nslice], zsem).wait()
        plsc.subcore_barrier()

        def fire_gathers(m, h):
            def gb(j, _):
                pltpu.async_copy(
                    tab_hbm.at[src_v.at[m * BKB + j]], rows_v.at[h, j], gsems[h]
                )
                return 0
            lax.fori_loop(0, BKB, gb, 0)

        def fire_scatters(m, h):
            def sb(j, _):
                pltpu.async_copy(
                    rows_v.at[h, j], acc.at[dst_v.at[m * BKB + j]],
                    ssems[h], add=True,
                )
                return 0
            lax.fori_loop(0, BKB, sb, 0)

        def drain(sem):
            def db(j, _):
                pltpu.make_async_copy(
                    tab_hbm.at[pl.ds(0, CHUNK)], rows_v.at[0, 0], sem
                ).wait()
                return 0
            lax.fori_loop(0, BKB, db, 0)

        for h in range(NH):
            fire_gathers(h, h)
        for m in range(NBATCH):
            h = m % NH
            drain(gsems[h])
            fire_scatters(m, h)
            drain(ssems[h])
            if m + NH < NBATCH:
                fire_gathers(m + NH, h)
        plsc.subcore_barrier()
        pltpu.sync_copy(acc.at[nslice], out_hbm.at[c, nslice])

    return k(table, src2d, dst2d, zeros16)


def _tc_pre(x, W1, deg_a, deg_b):
    def body(x_ref, w_ref, da_ref, db_ref, hp_ref, dinv_ref):
        dinv = lax.rsqrt(da_ref[...] + db_ref[...] + 1.0)
        h = jnp.dot(x_ref[...], w_ref[...], preferred_element_type=jnp.float32)
        hp_ref[pl.ds(0, N), :] = h * dinv[:N]
        hp_ref[pl.ds(N, N_PAD - N), :] = jnp.zeros((N_PAD - N, HID), jnp.float32)
        dinv_ref[...] = dinv

    return pl.pallas_call(
        body,
        out_shape=[
            jax.ShapeDtypeStruct((N_PAD, HID), jnp.float32),
            jax.ShapeDtypeStruct((N_PAD, 1), jnp.float32),
        ],
    )(x, W1, deg_a, deg_b)


def _tc_mid(s1a, s1b, hp1, dinv, b1):
    def body(sa_ref, sb_ref, hp_ref, di_ref, b_ref, g_ref):
        z = di_ref[...] * (sa_ref[...] + sb_ref[...] + hp_ref[...]) + b_ref[...]
        g_ref[...] = di_ref[...] * jnp.maximum(z, 0.0)

    return pl.pallas_call(
        body,
        out_shape=jax.ShapeDtypeStruct((N_PAD, HID), jnp.float32),
    )(s1a, s1b, hp1, dinv, b1)


def _tc_out(s2a, s2b, g, dinv, W2, b2):
    def body(sa_ref, sb_ref, g_ref, di_ref, w_ref, b_ref, out_ref):
        t = di_ref[...] * (sa_ref[...] + sb_ref[...] + g_ref[...])
        o = jnp.dot(t, w_ref[...], preferred_element_type=jnp.float32) + b_ref[...]
        m = jnp.max(o, axis=1, keepdims=True)
        lse = m + jnp.log(jnp.sum(jnp.exp(o - m), axis=1, keepdims=True))
        out_ref[...] = o - lse

    return pl.pallas_call(
        body,
        out_shape=jax.ShapeDtypeStruct((N_PAD, NCLASS), jnp.float32),
    )(s2a, s2b, g, dinv, W2, b2)


def kernel(x, adj, W1, b1, W2, b2):
    src, dst = adj[0], adj[1]
    pad_iota = lax.iota(jnp.int32, E_PAD - E)
    src2d = jnp.concatenate(
        [src, pad_iota % jnp.int32(N)]
    ).reshape(E_ROWS, CHUNK)
    dst2d = jnp.concatenate(
        [dst, jnp.int32(N) + pad_iota % jnp.int32(N_PAD - N)]
    ).reshape(E_ROWS, CHUNK)
    z16 = jnp.zeros((N_PAD, HID), jnp.float32)
    z1 = jnp.zeros((N_PAD,), jnp.float32)

    deg2 = _sc_degree(dst2d, z1)
    hp1, dinv = _tc_pre(x, W1, deg2[0][:, None], deg2[1][:, None])
    s1 = _sc_scatter16(hp1, src2d, dst2d, z16)
    g = _tc_mid(s1[0], s1[1], hp1, dinv, b1[None, :])
    s2 = _sc_scatter16(g, src2d, dst2d, z16)
    out = _tc_out(s2[0], s2[1], g, dinv, W2, b2[None, :])
    return out[:N]


# speedup vs baseline: 2.0964x; 1.2554x over previous
import functools

import jax
import jax.numpy as jnp
from jax import lax
from jax.experimental import pallas as pl
from jax.experimental.pallas import tpu as pltpu
from jax.experimental.pallas import tpu_sc as plsc

N = 10000
F_IN = 128
HID = 16
NCLASS = 40
E = 320000

CHUNK = 128
E_ROWS = 2560
E_PAD = E_ROWS * CHUNK
N_PAD = 10240
NUM_TILES = 32
ROWS_PER_TILE = E_ROWS // NUM_TILES
NROWS_PER_TILE = N_PAD // 16
NH = 4
BKB = 10
NBATCH = ROWS_PER_TILE // BKB


def _sc_mesh():
    return plsc.VectorSubcoreMesh(core_axis_name="c", subcore_axis_name="s")


def _sc_degree(sd2d, zeros1):
    @functools.partial(
        pl.kernel,
        mesh=_sc_mesh(),
        compiler_params=pltpu.CompilerParams(use_tc_tiling_on_sc=False),
        out_type=jax.ShapeDtypeStruct((2, N_PAD), jnp.float32),
        scratch_types=[
            pltpu.VMEM((ROWS_PER_TILE, CHUNK), jnp.int32),
            pltpu.VMEM((CHUNK,), jnp.float32),
            pltpu.VMEM_SHARED((N_PAD,), jnp.float32),
            pltpu.SemaphoreType.DMA,
            pltpu.SemaphoreType.DMA,
        ],
    )
    def k(sd_hbm, z_hbm, out_hbm, idx_v, ones_v, acc, zsem, ssem):
        c = lax.axis_index("c")
        s = lax.axis_index("s")
        wid = s * 2 + c
        for i in range(CHUNK // 16):
            ones_v[pl.ds(i * 16, 16)] = jnp.ones((16,), jnp.float32)
        nslice = pl.ds(s * NROWS_PER_TILE, NROWS_PER_TILE)
        pltpu.async_copy(z_hbm.at[nslice], acc.at[nslice], zsem)
        pltpu.sync_copy(
            sd_hbm.at[1, pl.ds(wid * ROWS_PER_TILE, ROWS_PER_TILE)], idx_v
        )
        pltpu.make_async_copy(z_hbm.at[nslice], acc.at[nslice], zsem).wait()
        plsc.subcore_barrier()

        def fire(i, _):
            pltpu.async_copy(ones_v, acc.at[idx_v.at[i]], ssem, add=True)
            return 0

        lax.fori_loop(0, ROWS_PER_TILE, fire, 0)

        def drain(i, _):
            pltpu.make_async_copy(sd_hbm.at[0, 0], idx_v.at[0], ssem).wait()
            return 0

        lax.fori_loop(0, ROWS_PER_TILE, drain, 0)
        plsc.subcore_barrier()
        pltpu.sync_copy(acc.at[nslice], out_hbm.at[c, nslice])

    return k(sd2d, zeros1)


def _sc_scatter16(table, sd2d, zeros16):
    @functools.partial(
        pl.kernel,
        mesh=_sc_mesh(),
        compiler_params=pltpu.CompilerParams(use_tc_tiling_on_sc=False),
        out_type=jax.ShapeDtypeStruct((2, N_PAD, HID), jnp.float32),
        scratch_types=[
            pltpu.VMEM((ROWS_PER_TILE, CHUNK), jnp.int32),
            pltpu.VMEM((ROWS_PER_TILE, CHUNK), jnp.int32),
            pltpu.VMEM((NH, BKB, CHUNK, HID), jnp.float32),
            pltpu.VMEM_SHARED((N_PAD, HID), jnp.float32),
            pltpu.SemaphoreType.DMA,
            [pltpu.SemaphoreType.DMA] * NH,
            [pltpu.SemaphoreType.DMA] * NH,
        ],
    )
    def k(tab_hbm, sd_hbm, z_hbm, out_hbm,
          src_v, dst_v, rows_v, acc, zsem, gsems, ssems):
        c = lax.axis_index("c")
        s = lax.axis_index("s")
        wid = s * 2 + c
        eslice = pl.ds(wid * ROWS_PER_TILE, ROWS_PER_TILE)
        nslice = pl.ds(s * NROWS_PER_TILE, NROWS_PER_TILE)
        pltpu.async_copy(z_hbm.at[nslice], acc.at[nslice], zsem)
        pltpu.sync_copy(sd_hbm.at[0, eslice], src_v)
        pltpu.sync_copy(sd_hbm.at[1, eslice], dst_v)
        pltpu.make_async_copy(z_hbm.at[nslice], acc.at[nslice], zsem).wait()
        plsc.subcore_barrier()

        def fire_gathers(m, h):
            def gb(j, _):
                pltpu.async_copy(
                    tab_hbm.at[src_v.at[m * BKB + j]], rows_v.at[h, j], gsems[h]
                )
                return 0
            lax.fori_loop(0, BKB, gb, 0)

        def fire_scatters(m, h):
            def sb(j, _):
                pltpu.async_copy(
                    rows_v.at[h, j], acc.at[dst_v.at[m * BKB + j]],
                    ssems[h], add=True,
                )
                return 0
            lax.fori_loop(0, BKB, sb, 0)

        def drain(sem):
            def db(j, _):
                pltpu.make_async_copy(
                    tab_hbm.at[pl.ds(0, CHUNK)], rows_v.at[0, 0], sem
                ).wait()
                return 0
            lax.fori_loop(0, BKB, db, 0)

        for h in range(NH):
            fire_gathers(h, h)
        for m in range(NBATCH):
            h = m % NH
            drain(gsems[h])
            fire_scatters(m, h)
            drain(ssems[h])
            if m + NH < NBATCH:
                fire_gathers(m + NH, h)
        plsc.subcore_barrier()
        pltpu.sync_copy(acc.at[nslice], out_hbm.at[c, nslice])

    return k(table, sd2d, zeros16)


def _dinv(deg_ref):
    d = deg_ref[0, :] + deg_ref[1, :] + 1.0
    return lax.rsqrt(d).reshape(N_PAD, 1)


def _tc_pre(x, W1, deg2):
    def body(x_ref, w_ref, d_ref, hp_ref):
        dinv = _dinv(d_ref)
        h = jnp.dot(x_ref[...], w_ref[...], preferred_element_type=jnp.float32)
        hp_ref[pl.ds(0, N), :] = h * dinv[:N]
        hp_ref[pl.ds(N, N_PAD - N), :] = jnp.zeros((N_PAD - N, HID), jnp.float32)

    return pl.pallas_call(
        body,
        out_shape=jax.ShapeDtypeStruct((N_PAD, HID), jnp.float32),
    )(x, W1, deg2)


def _tc_mid(s1, hp1, deg2, b1):
    def body(s_ref, hp_ref, d_ref, b_ref, g_ref):
        dinv = _dinv(d_ref)
        z = dinv * (s_ref[0] + s_ref[1] + hp_ref[...]) + b_ref[...]
        g_ref[...] = dinv * jnp.maximum(z, 0.0)

    return pl.pallas_call(
        body,
        out_shape=jax.ShapeDtypeStruct((N_PAD, HID), jnp.float32),
    )(s1, hp1, deg2, b1)


def _tc_out(s2, g, deg2, W2, b2):
    def body(s_ref, g_ref, d_ref, w_ref, b_ref, out_ref):
        dinv = _dinv(d_ref)
        t = dinv * (s_ref[0] + s_ref[1] + g_ref[...])
        o = jnp.dot(
            t[:N], w_ref[...], preferred_element_type=jnp.float32
        ) + b_ref[...]
        m = jnp.max(o, axis=1, keepdims=True)
        lse = m + jnp.log(jnp.sum(jnp.exp(o - m), axis=1, keepdims=True))
        out_ref[...] = o - lse

    return pl.pallas_call(
        body,
        out_shape=jax.ShapeDtypeStruct((N, NCLASS), jnp.float32),
    )(s2, g, deg2, W2, b2)


def kernel(x, adj, W1, b1, W2, b2):
    pad_iota = lax.iota(jnp.int32, E_PAD - E)
    pads = jnp.stack(
        [pad_iota % jnp.int32(N), jnp.int32(N) + pad_iota % jnp.int32(N_PAD - N)]
    )
    sd2d = jnp.concatenate([adj, pads], axis=1).reshape(2, E_ROWS, CHUNK)
    z16 = jnp.zeros((N_PAD, HID), jnp.float32)
    z1 = jnp.zeros((N_PAD,), jnp.float32)

    deg2 = _sc_degree(sd2d, z1)
    hp1 = _tc_pre(x, W1, deg2)
    s1 = _sc_scatter16(hp1, sd2d, z16)
    g = _tc_mid(s1, hp1, deg2, b1[None, :])
    s2 = _sc_scatter16(g, sd2d, z16)
    return _tc_out(s2, g, deg2, W2, b2[None, :])


# speedup vs baseline: 2.5572x; 1.2198x over previous
import functools

import jax
import jax.numpy as jnp
from jax import lax
from jax.experimental import pallas as pl
from jax.experimental.pallas import tpu as pltpu
from jax.experimental.pallas import tpu_sc as plsc

N = 10000
F_IN = 128
HID = 16
NCLASS = 40
E = 320000

CHUNK = 128
E_ROWS = 2560
E_PAD = E_ROWS * CHUNK
N_PAD = 10240
NUM_TILES = 32
ROWS_PER_TILE = E_ROWS // NUM_TILES
NROWS_PER_TILE = N_PAD // 16
NH = 4
BKB = 10
NBATCH = ROWS_PER_TILE // BKB


def _sc_mesh():
    return plsc.VectorSubcoreMesh(core_axis_name="c", subcore_axis_name="s")


def _sc_degree(sd2d, zeros1):
    @functools.partial(
        pl.kernel,
        mesh=_sc_mesh(),
        compiler_params=pltpu.CompilerParams(use_tc_tiling_on_sc=False),
        out_type=jax.ShapeDtypeStruct((2, N_PAD), jnp.float32),
        scratch_types=[
            pltpu.VMEM((ROWS_PER_TILE, CHUNK), jnp.int32),
            pltpu.VMEM((CHUNK,), jnp.float32),
            pltpu.VMEM_SHARED((N_PAD,), jnp.float32),
            pltpu.SemaphoreType.DMA,
            pltpu.SemaphoreType.DMA,
        ],
    )
    def k(sd_hbm, z_hbm, out_hbm, idx_v, ones_v, acc, zsem, ssem):
        c = lax.axis_index("c")
        s = lax.axis_index("s")
        wid = s * 2 + c
        for i in range(CHUNK // 16):
            ones_v[pl.ds(i * 16, 16)] = jnp.ones((16,), jnp.float32)
        nslice = pl.ds(s * NROWS_PER_TILE, NROWS_PER_TILE)
        pltpu.async_copy(z_hbm.at[nslice], acc.at[nslice], zsem)
        pltpu.sync_copy(
            sd_hbm.at[1, pl.ds(wid * ROWS_PER_TILE, ROWS_PER_TILE)], idx_v
        )
        pltpu.make_async_copy(z_hbm.at[nslice], acc.at[nslice], zsem).wait()
        plsc.subcore_barrier()

        def fire(i, _):
            pltpu.async_copy(ones_v, acc.at[idx_v.at[i]], ssem, add=True)
            return 0

        lax.fori_loop(0, ROWS_PER_TILE, fire, 0)

        def drain(i, _):
            pltpu.make_async_copy(sd_hbm.at[0, 0], idx_v.at[0], ssem).wait()
            return 0

        lax.fori_loop(0, ROWS_PER_TILE, drain, 0)
        plsc.subcore_barrier()
        pltpu.sync_copy(acc.at[nslice], out_hbm.at[c, nslice])

    return k(sd2d, zeros1)


def _sc_scatter16(table, sd2d, zeros16):
    @functools.partial(
        pl.kernel,
        mesh=_sc_mesh(),
        compiler_params=pltpu.CompilerParams(use_tc_tiling_on_sc=False),
        out_type=jax.ShapeDtypeStruct((2, N_PAD, HID), jnp.float32),
        scratch_types=[
            pltpu.VMEM((ROWS_PER_TILE, CHUNK), jnp.int32),
            pltpu.VMEM((ROWS_PER_TILE, CHUNK), jnp.int32),
            pltpu.VMEM((NH, BKB, CHUNK, HID), jnp.float32),
            pltpu.VMEM_SHARED((N_PAD, HID), jnp.float32),
            pltpu.SemaphoreType.DMA,
            [pltpu.SemaphoreType.DMA] * NH,
            [pltpu.SemaphoreType.DMA] * NH,
        ],
    )
    def k(tab_hbm, sd_hbm, z_hbm, out_hbm,
          src_v, dst_v, rows_v, acc, zsem, gsems, ssems):
        c = lax.axis_index("c")
        s = lax.axis_index("s")
        wid = s * 2 + c
        eslice = pl.ds(wid * ROWS_PER_TILE, ROWS_PER_TILE)
        nslice = pl.ds(s * NROWS_PER_TILE, NROWS_PER_TILE)
        pltpu.async_copy(z_hbm.at[nslice], acc.at[nslice], zsem)
        pltpu.sync_copy(sd_hbm.at[0, eslice], src_v)
        pltpu.sync_copy(sd_hbm.at[1, eslice], dst_v)
        pltpu.make_async_copy(z_hbm.at[nslice], acc.at[nslice], zsem).wait()
        plsc.subcore_barrier()

        def fire_gathers(m, h):
            def gb(j, _):
                pltpu.async_copy(
                    tab_hbm.at[src_v.at[m * BKB + j]], rows_v.at[h, j], gsems[h]
                )
                return 0
            lax.fori_loop(0, BKB, gb, 0)

        def fire_scatters(m, h):
            def sb(j, _):
                pltpu.async_copy(
                    rows_v.at[h, j], acc.at[dst_v.at[m * BKB + j]],
                    ssems[h], add=True,
                )
                return 0
            lax.fori_loop(0, BKB, sb, 0)

        def drain(sem):
            def db(j, _):
                pltpu.make_async_copy(
                    tab_hbm.at[pl.ds(0, CHUNK)], rows_v.at[0, 0], sem
                ).wait()
                return 0
            lax.fori_loop(0, BKB, db, 0)

        for h in range(NH):
            fire_gathers(h, h)
        for m in range(NBATCH):
            h = m % NH
            drain(gsems[h])
            fire_scatters(m, h)
            drain(ssems[h])
            if m + NH < NBATCH:
                fire_gathers(m + NH, h)
        plsc.subcore_barrier()
        pltpu.sync_copy(acc.at[nslice], out_hbm.at[c, nslice])

    return k(table, sd2d, zeros16)


NF = N_PAD * HID // 128
NFR = N * HID // 128


def _tc_pre(xf, W1f, deg3, R8):
    def body(x_ref, w_ref, d_ref, r_ref, hp_ref, df_ref):
        q = lax.rsqrt(d_ref[0] + d_ref[1] + 1.0)
        df = jnp.dot(q, r_ref[...], preferred_element_type=jnp.float32)
        h = jnp.dot(x_ref[...], w_ref[...], preferred_element_type=jnp.float32)
        hp_ref[...] = h * df
        df_ref[...] = df

    return pl.pallas_call(
        body,
        out_shape=[
            jax.ShapeDtypeStruct((NF, 128), jnp.float32),
            jax.ShapeDtypeStruct((NF, 128), jnp.float32),
        ],
    )(xf, W1f, deg3, R8)


def _tc_mid(s1f, hp1f, dinvf, b1f):
    def body(s_ref, hp_ref, df_ref, b_ref, g_ref):
        df = df_ref[...]
        z = df * (s_ref[0] + s_ref[1] + hp_ref[...]) + b_ref[...]
        g_ref[...] = df * jnp.maximum(z, 0.0)

    return pl.pallas_call(
        body,
        out_shape=jax.ShapeDtypeStruct((NF, 128), jnp.float32),
    )(s1f, hp1f, dinvf, b1f)


def _tc_out(s2f, gf, dinvf, W2f, b2f):
    def body(s_ref, g_ref, df_ref, w_ref, b_ref, out_ref):
        tf = df_ref[...] * (s_ref[0] + s_ref[1] + g_ref[...])
        z = jnp.dot(
            tf, w_ref[...], preferred_element_type=jnp.float32
        ) + b_ref[...]
        outs = []
        for b in range(8):
            zb = z[:, b * NCLASS:(b + 1) * NCLASS]
            mb = jnp.max(zb, axis=1, keepdims=True)
            eb = jnp.exp(zb - mb)
            lb = mb + jnp.log(jnp.sum(eb, axis=1, keepdims=True))
            outs.append(zb - lb)
        out_ref[...] = jnp.concatenate(outs, axis=1)

    return pl.pallas_call(
        body,
        out_shape=jax.ShapeDtypeStruct((NF, 8 * NCLASS), jnp.float32),
    )(s2f, gf, dinvf, W2f, b2f)


def kernel(x, adj, W1, b1, W2, b2):
    pad_iota = lax.iota(jnp.int32, E_PAD - E)
    pads = jnp.stack(
        [pad_iota % jnp.int32(N), jnp.int32(N) + pad_iota % jnp.int32(N_PAD - N)]
    )
    sd2d = jnp.concatenate([adj, pads], axis=1).reshape(2, E_ROWS, CHUNK)
    z16 = jnp.zeros((N_PAD, HID), jnp.float32)
    z1 = jnp.zeros((N_PAD,), jnp.float32)

    eye8 = jnp.eye(8, dtype=jnp.float32)
    W1f = jnp.kron(eye8, W1)
    W2f = jnp.kron(eye8, W2)
    R8 = jnp.kron(eye8, jnp.ones((1, HID), jnp.float32))
    b1f = jnp.tile(b1, 8)[None, :]
    b2f = jnp.tile(b2, 8)[None, :]
    xf = jnp.pad(x.reshape(NFR, 8 * F_IN), ((0, NF - NFR), (0, 0)))

    deg2 = _sc_degree(sd2d, z1)
    hp1f, dinvf = _tc_pre(xf, W1f, deg2.reshape(2, NF, 8), R8)
    s1 = _sc_scatter16(hp1f.reshape(N_PAD, HID), sd2d, z16)
    gf = _tc_mid(s1.reshape(2, NF, 128), hp1f, dinvf, b1f)
    s2 = _sc_scatter16(gf.reshape(N_PAD, HID), sd2d, z16)
    of = _tc_out(s2.reshape(2, NF, 128), gf, dinvf, W2f, b2f)
    return of[:NFR].reshape(N, NCLASS)


# speedup vs baseline: 2.6199x; 1.0245x over previous
import functools

import jax
import jax.numpy as jnp
from jax import lax
from jax.experimental import pallas as pl
from jax.experimental.pallas import tpu as pltpu
from jax.experimental.pallas import tpu_sc as plsc

N = 10000
F_IN = 128
HID = 16
NCLASS = 40
E = 320000

CHUNK = 128
E_ROWS = 2560
E_PAD = E_ROWS * CHUNK
N_PAD = 10240
NUM_TILES = 32
ROWS_PER_TILE = E_ROWS // NUM_TILES
NROWS_PER_TILE = N_PAD // 16
NH = 4
BKB = 10
NBATCH = ROWS_PER_TILE // BKB


def _sc_mesh():
    return plsc.VectorSubcoreMesh(core_axis_name="c", subcore_axis_name="s")


def _sc_degree(sd2d, zeros1):
    @functools.partial(
        pl.kernel,
        mesh=_sc_mesh(),
        compiler_params=pltpu.CompilerParams(use_tc_tiling_on_sc=False),
        out_type=jax.ShapeDtypeStruct((2, N_PAD), jnp.float32),
        scratch_types=[
            pltpu.VMEM((ROWS_PER_TILE, CHUNK), jnp.int32),
            pltpu.VMEM((CHUNK,), jnp.float32),
            pltpu.VMEM_SHARED((N_PAD,), jnp.float32),
            pltpu.SemaphoreType.DMA,
            pltpu.SemaphoreType.DMA,
        ],
    )
    def k(sd_hbm, z_hbm, out_hbm, idx_v, ones_v, acc, zsem, ssem):
        c = lax.axis_index("c")
        s = lax.axis_index("s")
        wid = s * 2 + c
        for i in range(CHUNK // 16):
            ones_v[pl.ds(i * 16, 16)] = jnp.ones((16,), jnp.float32)
        nslice = pl.ds(s * NROWS_PER_TILE, NROWS_PER_TILE)
        pltpu.async_copy(z_hbm.at[nslice], acc.at[nslice], zsem)
        pltpu.sync_copy(
            sd_hbm.at[1, pl.ds(wid * ROWS_PER_TILE, ROWS_PER_TILE)], idx_v
        )
        pltpu.make_async_copy(z_hbm.at[nslice], acc.at[nslice], zsem).wait()
        plsc.subcore_barrier()

        def fire(i, _):
            pltpu.async_copy(ones_v, acc.at[idx_v.at[i]], ssem, add=True)
            return 0

        lax.fori_loop(0, ROWS_PER_TILE, fire, 0)

        def drain(i, _):
            pltpu.make_async_copy(sd_hbm.at[0, 0], idx_v.at[0], ssem).wait()
            return 0

        lax.fori_loop(0, ROWS_PER_TILE, drain, 0)
        plsc.subcore_barrier()
        pltpu.sync_copy(acc.at[nslice], out_hbm.at[c, nslice])

    return k(sd2d, zeros1)


def _sc_scatter16(table, sd2d, zeros16):
    @functools.partial(
        pl.kernel,
        mesh=_sc_mesh(),
        compiler_params=pltpu.CompilerParams(use_tc_tiling_on_sc=False),
        out_type=jax.ShapeDtypeStruct((2, N_PAD, HID), jnp.float32),
        scratch_types=[
            pltpu.VMEM((ROWS_PER_TILE, CHUNK), jnp.int32),
            pltpu.VMEM((ROWS_PER_TILE, CHUNK), jnp.int32),
            pltpu.VMEM((NH, BKB, CHUNK, HID), jnp.float32),
            pltpu.VMEM_SHARED((N_PAD, HID), jnp.float32),
            pltpu.SemaphoreType.DMA,
            [pltpu.SemaphoreType.DMA] * NH,
            [pltpu.SemaphoreType.DMA] * NH,
        ],
    )
    def k(tab_hbm, sd_hbm, z_hbm, out_hbm,
          src_v, dst_v, rows_v, acc, zsem, gsems, ssems):
        c = lax.axis_index("c")
        s = lax.axis_index("s")
        wid = s * 2 + c
        eslice = pl.ds(wid * ROWS_PER_TILE, ROWS_PER_TILE)
        nslice = pl.ds(s * NROWS_PER_TILE, NROWS_PER_TILE)
        pltpu.async_copy(z_hbm.at[nslice], acc.at[nslice], zsem)
        pltpu.sync_copy(sd_hbm.at[0, eslice], src_v)
        pltpu.sync_copy(sd_hbm.at[1, eslice], dst_v)
        pltpu.make_async_copy(z_hbm.at[nslice], acc.at[nslice], zsem).wait()
        plsc.subcore_barrier()

        def fire_gathers(m, h):
            def gb(j, _):
                pltpu.async_copy(
                    tab_hbm.at[src_v.at[m * BKB + j]], rows_v.at[h, j], gsems[h]
                )
                return 0
            lax.fori_loop(0, BKB, gb, 0)

        def fire_scatters(m, h):
            def sb(j, _):
                pltpu.async_copy(
                    rows_v.at[h, j], acc.at[dst_v.at[m * BKB + j]],
                    ssems[h], add=True,
                )
                return 0
            lax.fori_loop(0, BKB, sb, 0)

        def drain(sem):
            def db(j, _):
                pltpu.make_async_copy(
                    tab_hbm.at[pl.ds(0, CHUNK)], rows_v.at[0, 0], sem
                ).wait()
                return 0
            lax.fori_loop(0, BKB, db, 0)

        for h in range(NH):
            fire_gathers(h, h)
        for m in range(NBATCH):
            h = m % NH
            drain(gsems[h])
            fire_scatters(m, h)
            drain(ssems[h])
            if m + NH < NBATCH:
                fire_gathers(m + NH, h)
        plsc.subcore_barrier()
        pltpu.sync_copy(acc.at[nslice], out_hbm.at[c, nslice])

    return k(table, sd2d, zeros16)


NF = N_PAD * HID // 128
NFR = N * HID // 128


def _tc_pre(xf, W1f, deg3, R8):
    def body(x_ref, w_ref, d_ref, r_ref, hp_ref, df_ref):
        q = lax.rsqrt(d_ref[0] + d_ref[1] + 1.0)
        df = jnp.dot(q, r_ref[...], preferred_element_type=jnp.float32)
        h = jnp.dot(x_ref[...], w_ref[...], preferred_element_type=jnp.float32)
        hp_ref[...] = h * df
        df_ref[...] = df

    return pl.pallas_call(
        body,
        out_shape=[
            jax.ShapeDtypeStruct((NF, 128), jnp.float32),
            jax.ShapeDtypeStruct((NF, 128), jnp.float32),
        ],
    )(xf, W1f, deg3, R8)


def _tc_mid(s1f, hp1f, dinvf, b1f):
    def body(s_ref, hp_ref, df_ref, b_ref, g_ref):
        df = df_ref[...]
        z = df * (s_ref[0] + s_ref[1] + hp_ref[...]) + b_ref[...]
        g_ref[...] = df * jnp.maximum(z, 0.0)

    return pl.pallas_call(
        body,
        out_shape=jax.ShapeDtypeStruct((NF, 128), jnp.float32),
    )(s1f, hp1f, dinvf, b1f)


def _tc_out(s2f, gf, dinvf, W2f, b2f):
    def body(s_ref, g_ref, df_ref, w_ref, b_ref, out_ref):
        tf = df_ref[...] * (s_ref[0] + s_ref[1] + g_ref[...])
        z = jnp.dot(
            tf, w_ref[...], preferred_element_type=jnp.float32
        ) + b_ref[...]
        outs = []
        for b in range(8):
            zb = z[:, b * NCLASS:(b + 1) * NCLASS]
            mb = jnp.max(zb, axis=1, keepdims=True)
            eb = jnp.exp(zb - mb)
            lb = mb + jnp.log(jnp.sum(eb, axis=1, keepdims=True))
            outs.append(zb - lb)
        out_ref[...] = jnp.concatenate(outs, axis=1)[:NFR]

    return pl.pallas_call(
        body,
        out_shape=jax.ShapeDtypeStruct((NFR, 8 * NCLASS), jnp.float32),
    )(s2f, gf, dinvf, W2f, b2f)


def kernel(x, adj, W1, b1, W2, b2):
    pad_iota = lax.iota(jnp.int32, E_PAD - E)
    pads = jnp.stack(
        [pad_iota % jnp.int32(N), jnp.int32(N) + pad_iota % jnp.int32(N_PAD - N)]
    )
    sd2d = jnp.concatenate([adj, pads], axis=1).reshape(2, E_ROWS, CHUNK)
    z16 = jnp.zeros((N_PAD, HID), jnp.float32)
    z1 = jnp.zeros((N_PAD,), jnp.float32)

    eye8 = jnp.eye(8, dtype=jnp.float32)
    W1f = jnp.kron(eye8, W1)
    W2f = jnp.kron(eye8, W2)
    R8 = jnp.kron(eye8, jnp.ones((1, HID), jnp.float32))
    b1f = jnp.tile(b1, 8)[None, :]
    b2f = jnp.tile(b2, 8)[None, :]
    xf = jnp.pad(x.reshape(NFR, 8 * F_IN), ((0, NF - NFR), (0, 0)))

    deg2 = _sc_degree(sd2d, z1)
    hp1f, dinvf = _tc_pre(xf, W1f, deg2.reshape(2, NF, 8), R8)
    s1 = _sc_scatter16(hp1f.reshape(N_PAD, HID), sd2d, z16)
    gf = _tc_mid(s1.reshape(2, NF, 128), hp1f, dinvf, b1f)
    s2 = _sc_scatter16(gf.reshape(N_PAD, HID), sd2d, z16)
    of = _tc_out(s2.reshape(2, NF, 128), gf, dinvf, W2f, b2f)
    return of.reshape(N, NCLASS)
